# baseline (speedup 1.0000x reference)
import functools

import jax
import jax.numpy as jnp
from jax import lax
from jax.experimental import pallas as pl
from jax.experimental.pallas import tpu as pltpu
from jax.experimental.pallas import tpu_sc as plsc

CH = 64
EPS = 1e-5
_NC = 2
_NS = 16
_W = _NC * _NS

_RB = 2
_GI = 40


def _rsqrt(v):
    i = lax.bitcast_convert_type(v, jnp.int32)
    i = jnp.int32(0x5F3759DF) - lax.shift_right_arithmetic(i, 1)
    y = lax.bitcast_convert_type(i, jnp.float32)
    for _ in range(2):
        y = y * (1.5 - 0.5 * v * y * y)
    return y


def _ln_loop(buf, g4, b4, nrows, m_sc, r_sc, tmp=None):
    @plsc.parallel_loop(0, nrows, unroll=8)
    def _(r):
        a = [buf[r, pl.ds(16 * i, 16)] for i in range(4)]
        s = (a[0] + a[1]) + (a[2] + a[3])
        q = (a[0] * a[0] + a[1] * a[1]) + (a[2] * a[2] + a[3] * a[3])
        mean = jnp.sum(s) * (1.0 / CH)
        var = jnp.sum(q) * (1.0 / CH) - mean * mean
        m_sc[r] = mean
        r_sc[r] = _rsqrt(var + EPS)

    @plsc.parallel_loop(0, nrows, unroll=4)
    def _(r):
        m = m_sc[r]
        rs = r_sc[r]
        for i in range(4):
            val = (buf[r, pl.ds(16 * i, 16)] - m) * (rs * g4[i]) + b4[i]
            if tmp is not None:
                val = val + tmp[r, pl.ds(16 * i, 16)]
            buf[r, pl.ds(16 * i, 16)] = val


def _wid():
    return lax.axis_index("s") * _NC + lax.axis_index("c")


def _tmp_norm_body(tbl_hbm, g_hbm, b_hbm, out_hbm, buf, g_v, b_v, m_sc, r_sc):
    rows = tbl_hbm.shape[0] // _W
    base = _wid() * rows
    pltpu.sync_copy(g_hbm, g_v)
    pltpu.sync_copy(b_hbm, b_v)
    pltpu.sync_copy(tbl_hbm.at[pl.ds(base, rows)], buf)
    g4 = [g_v[pl.ds(16 * i, 16)] for i in range(4)]
    b4 = [b_v[pl.ds(16 * i, 16)] for i in range(4)]
    _ln_loop(buf, g4, b4, rows, m_sc, r_sc)
    pltpu.sync_copy(buf, out_hbm.at[pl.ds(base, rows)])


def _main_body(nchunk, seq, tok_hbm, x_hbm, t_hbm, ntmp_hbm, g_hbm, b_hbm,
               out_hbm, xb0, xb1, tb0, tb1, tok0, tok1, tmp0, tmp1, g_v, b_v,
               m_sc, r_sc, si0, si1, sg0, sg1, so0, so1):
    wid = _wid()
    pltpu.sync_copy(g_hbm, g_v)
    pltpu.sync_copy(b_hbm, b_v)
    g4 = [g_v[pl.ds(16 * i, 16)] for i in range(4)]
    b4 = [b_v[pl.ds(16 * i, 16)] for i in range(4)]

    xb = [xb0, xb1]
    tb = [tb0, tb1]
    tok = [tok0, tok1]
    tmp = [tmp0, tmp1]
    si = [si0, si1]
    sg = [sg0, sg1]
    so = [so0, so1]
    ng = seq // _GI

    bbase = wid * (nchunk * _RB)

    def issue_gathers(b):
        for j in range(_RB):
            for k in range(ng):
                idx_x = xb[b].at[j, pl.ds(k * _GI, _GI)]
                idx_t = tb[b].at[j, pl.ds(k * _GI, _GI)]
                dst = pl.ds(k * _GI, _GI)
                pltpu.async_copy(tok_hbm.at[idx_x], tok[b].at[j, dst], sg[b])
                pltpu.async_copy(ntmp_hbm.at[idx_t], tmp[b].at[j, dst], sg[b])

    def wait_gathers(b):
        pltpu.make_async_copy(out_hbm.at[pl.ds(0, _RB)], tok[b], sg[b]).wait()
        pltpu.make_async_copy(out_hbm.at[pl.ds(0, _RB)], tmp[b], sg[b]).wait()

    def issue_idx(h, b):
        row = bbase + h * _RB
        pltpu.async_copy(x_hbm.at[pl.ds(row, _RB)], xb[b], si[b])
        pltpu.async_copy(t_hbm.at[pl.ds(row, _RB)], tb[b], si[b])

    def wait_idx(b):
        pltpu.make_async_copy(x_hbm.at[pl.ds(0, _RB)], xb[b], si[b]).wait()
        pltpu.make_async_copy(t_hbm.at[pl.ds(0, _RB)], tb[b], si[b]).wait()

    def wait_out(b):
        pltpu.make_async_copy(out_hbm.at[pl.ds(0, _RB)], tok[b], so[b]).wait()

    pltpu.sync_copy(x_hbm.at[pl.ds(bbase, _RB)], xb[0])
    pltpu.sync_copy(t_hbm.at[pl.ds(bbase, _RB)], tb[0])
    issue_gathers(0)
    issue_idx(1, 1)

    def half(g, b):
        nb = 1 - b
        wait_gathers(b)
        issue_idx(jnp.minimum(g + 2, nchunk - 1), b)
        wait_idx(nb)

        @pl.when(g >= 1)
        def _():
            wait_out(nb)
        issue_gathers(nb)
        for j in range(_RB):
            _ln_loop(tok[b].at[j], g4, b4, seq, m_sc, r_sc, tmp=tmp[b].at[j])
        pltpu.async_copy(tok[b], out_hbm.at[pl.ds(bbase + g * _RB, _RB)],
                         so[b])

    def pair(p, carry):
        half(2 * p, 0)
        half(2 * p + 1, 1)
        return carry

    lax.fori_loop(0, nchunk // 2, pair, 0)

    wait_idx(1)
    wait_gathers(0)
    wait_out(1)


def kernel(x, t, pad, token_table, tok_gamma, tok_beta, temporal_table,
           tmp_gamma, tmp_beta):
    del pad
    bsz, seq = x.shape
    assert seq % _GI == 0 and bsz % (_W * _RB * 2) == 0
    nchunk = bsz // (_W * _RB)
    xi = x.astype(jnp.int32)
    ti = t.astype(jnp.int32)
    mesh = plsc.VectorSubcoreMesh(core_axis_name="c", subcore_axis_name="s")
    params = pltpu.CompilerParams(
        needs_layout_passes=False, use_tc_tiling_on_sc=False)

    tmp_norm = pl.kernel(
        _tmp_norm_body,
        out_type=jax.ShapeDtypeStruct(temporal_table.shape, jnp.float32),
        mesh=mesh,
        compiler_params=params,
        scratch_types=[
            pltpu.VMEM((temporal_table.shape[0] // _W, CH), jnp.float32),
            pltpu.VMEM((CH,), jnp.float32),
            pltpu.VMEM((CH,), jnp.float32),
            pltpu.SMEM((temporal_table.shape[0] // _W,), jnp.float32),
            pltpu.SMEM((temporal_table.shape[0] // _W,), jnp.float32),
        ],
    )
    ntmp = tmp_norm(temporal_table, tmp_gamma, tmp_beta)

    main = pl.kernel(
        functools.partial(_main_body, nchunk, seq),
        out_type=jax.ShapeDtypeStruct((bsz, seq, CH), jnp.float32),
        mesh=mesh,
        compiler_params=params,
        scratch_types=(
            [pltpu.VMEM((_RB, seq), jnp.int32)] * 4
            + [pltpu.VMEM((_RB, seq, CH), jnp.float32)] * 4
            + [pltpu.VMEM((CH,), jnp.float32)] * 2
            + [pltpu.SMEM((seq,), jnp.float32)] * 2
            + [pltpu.SemaphoreType.DMA] * 6
        ),
    )
    return main(token_table, xi, ti, ntmp, tok_gamma, tok_beta)
